# baseline (speedup 1.0000x reference)
import functools

import jax
import jax.numpy as jnp
from jax import lax
from jax.experimental import pallas as pl
from jax.experimental.pallas import tpu as pltpu
from jax.experimental.pallas import tpu_sc as plsc

_PADF = 32


@functools.lru_cache(maxsize=None)
def _build(B, F, D):
    info = plsc.get_sparse_core_info()
    NC, NS = info.num_cores, info.num_subcores
    NW = NC * NS
    assert B % NW == 0
    b_per_w = B // NW
    mesh = plsc.VectorSubcoreMesh(core_axis_name="c", subcore_axis_name="s")

    half = b_per_w // 2

    @functools.partial(
        pl.kernel,
        mesh=mesh,
        out_type=jax.ShapeDtypeStruct((B, _PADF, D), jnp.float32),
        scratch_types=[
            pltpu.VMEM((half, _PADF), jnp.int32),
            pltpu.VMEM((half, _PADF, D), jnp.float32),
            pltpu.SemaphoreType.DMA,
        ],
        compiler_params=pltpu.CompilerParams(use_tc_tiling_on_sc=False),
    )
    def k(idx_hbm, table_hbm, out_hbm, idx_v, rows_v, sem):
        wid = lax.axis_index("s") * NC + lax.axis_index("c")
        for h in range(2):
            base = wid * b_per_w + h * half
            pltpu.sync_copy(idx_hbm.at[pl.ds(base, half), :], idx_v)
            copies = [
                pltpu.async_copy(table_hbm.at[idx_v.at[j]], rows_v.at[j], sem)
                for j in range(half)
            ]
            for c in copies:
                c.wait()
            pltpu.sync_copy(rows_v, out_hbm.at[pl.ds(base, half)])

    return k


def _pad_body(x_ref, o_ref):
    o_ref[...] = jnp.pad(
        x_ref[...], ((0, 0), (0, o_ref.shape[1] - x_ref.shape[1]))
    )


@functools.lru_cache(maxsize=None)
def _pad(B, F):
    return pl.pallas_call(
        _pad_body,
        out_shape=jax.ShapeDtypeStruct((B, _PADF), jnp.int32),
    )


def kernel(x, weight):
    B, F = x.shape
    D = weight.shape[1]
    idx = _pad(B, F)(x.astype(jnp.int32))
    out_p = _build(B, F, D)(idx, weight)
    return out_p[:, :F, :]


# speedup vs baseline: 1.4266x; 1.4266x over previous
import functools

import jax
import jax.numpy as jnp
from jax import lax
from jax.experimental import pallas as pl
from jax.experimental.pallas import tpu as pltpu
from jax.experimental.pallas import tpu_sc as plsc

_LANES = 128
_PADF = 32


@functools.lru_cache(maxsize=None)
def _build(B, F, D):
    info = plsc.get_sparse_core_info()
    NC, NS = info.num_cores, info.num_subcores
    NW = NC * NS
    assert B % NW == 0
    b_per_w = B // NW
    half = b_per_w // 2
    mesh = plsc.VectorSubcoreMesh(core_axis_name="c", subcore_axis_name="s")

    @functools.partial(
        pl.kernel,
        mesh=mesh,
        out_type=jax.ShapeDtypeStruct((B, _PADF, D), jnp.float32),
        scratch_types=[
            pltpu.VMEM((half, _LANES), jnp.int32),
            pltpu.VMEM((half, _PADF, D), jnp.float32),
            pltpu.SemaphoreType.DMA,
        ],
        compiler_params=pltpu.CompilerParams(use_tc_tiling_on_sc=False),
    )
    def k(idx_hbm, table_hbm, out_hbm, idx_v, rows_v, sem):
        wid = lax.axis_index("s") * NC + lax.axis_index("c")
        for h in range(2):
            base = wid * b_per_w + h * half
            pltpu.sync_copy(idx_hbm.at[pl.ds(base, half), :], idx_v)
            copies = [
                pltpu.async_copy(
                    table_hbm.at[idx_v.at[j, pl.ds(0, _PADF)]],
                    rows_v.at[j],
                    sem,
                )
                for j in range(half)
            ]
            for c in copies:
                c.wait()
            pltpu.sync_copy(rows_v, out_hbm.at[pl.ds(base, half)])

    return k


def _widen_body(x_ref, o_ref):
    B, F = x_ref.shape
    xp = jnp.pad(x_ref[...], ((0, 0), (0, _LANES - F)))
    lane = jax.lax.broadcasted_iota(jnp.int32, (B, _LANES), 1)
    row = jax.lax.broadcasted_iota(jnp.int32, (B, _LANES), 0)
    o_ref[...] = jnp.where(lane < F, xp, row)


@functools.lru_cache(maxsize=None)
def _widen(B):
    return pl.pallas_call(
        _widen_body,
        out_shape=jax.ShapeDtypeStruct((B, _LANES), jnp.int32),
    )


def kernel(x, weight):
    B, F = x.shape
    D = weight.shape[1]
    idx = _widen(B)(x.astype(jnp.int32))
    out_p = _build(B, F, D)(idx, weight)
    return out_p[:, :F, :]


# speedup vs baseline: 1.4505x; 1.0168x over previous
import functools

import jax
import jax.numpy as jnp
from jax import lax
from jax.experimental import pallas as pl
from jax.experimental.pallas import tpu as pltpu
from jax.experimental.pallas import tpu_sc as plsc

_CHUNK = 128


@functools.lru_cache(maxsize=None)
def _build(B, D):
    info = plsc.get_sparse_core_info()
    NC, NS = info.num_cores, info.num_subcores
    NW = NC * NS
    assert B % (NW * _CHUNK) == 0
    b_per_w = B // NW
    n_chunks = b_per_w // _CHUNK
    mesh = plsc.VectorSubcoreMesh(core_axis_name="c", subcore_axis_name="s")

    @functools.partial(
        pl.kernel,
        mesh=mesh,
        out_type=jax.ShapeDtypeStruct((B, D), jnp.float32),
        scratch_types=[
            pltpu.VMEM((n_chunks, _CHUNK), jnp.int32),
            pltpu.VMEM((b_per_w, D), jnp.float32),
            pltpu.SemaphoreType.DMA,
        ],
        compiler_params=pltpu.CompilerParams(use_tc_tiling_on_sc=False),
    )
    def k(idx_hbm, table_hbm, out_hbm, idx_v, rows_v, sem):
        wid = lax.axis_index("s") * NC + lax.axis_index("c")
        pltpu.sync_copy(idx_hbm.at[pl.ds(wid * n_chunks, n_chunks), :], idx_v)
        copies = [
            pltpu.async_copy(
                table_hbm.at[idx_v.at[j]],
                rows_v.at[pl.ds(j * _CHUNK, _CHUNK), :],
                sem,
            )
            for j in range(n_chunks)
        ]
        for c in copies:
            c.wait()
        pltpu.sync_copy(rows_v, out_hbm.at[pl.ds(wid * b_per_w, b_per_w)])

    return k


def kernel(x, weight):
    B, F = x.shape
    D = weight.shape[1]
    n = B * F
    idx = x.astype(jnp.int32).reshape(n // _CHUNK, _CHUNK)
    out = _build(n, D)(idx, weight)
    return out.reshape(B, F, D)
